# baseline (speedup 1.0000x reference)
import jax
import jax.numpy as jnp
from jax.experimental import pallas as pl
from jax.experimental.pallas import tpu as pltpu

B = 8
C = 192
C2 = 2 * C
R = C2 // 16
E = 16
K = 2
TEMP = 2.0
S = 224 * 224
ROWS = B * C
H = 224
RB = 32
NRB = ROWS // RB
EPS = 1e-5


def _gelu(t):
    return 0.5 * t * (1.0 + jax.lax.erf(t * (1.0 / jnp.sqrt(2.0))))


NBUF = 4


def _rowsum_kernel(x_hbm, out_ref, *bufs):
    slabs = bufs[:NBUF]
    sems = bufs[NBUF:]
    handles = [None] * NRB
    for j in range(min(NBUF, NRB)):
        handles[j] = pltpu.make_async_copy(
            x_hbm.at[pl.ds(j * RB, RB)], slabs[j % NBUF], sems[j % NBUF])
        handles[j].start()
    for j in range(NRB):
        handles[j].wait()
        s = jnp.sum(slabs[j % NBUF][...], axis=1)
        out_ref[pl.ds(j * RB, RB), :] = jnp.sum(s, axis=1, keepdims=True)
        nxt = j + NBUF
        if nxt < NRB:
            handles[nxt] = pltpu.make_async_copy(
                x_hbm.at[pl.ds(nxt * RB, RB)], slabs[nxt % NBUF], sems[nxt % NBUF])
            handles[nxt].start()


def _mlp_kernel(g_ref, w1t_ref, b1_ref, bn1g_ref, bn1b_ref,
                caw1t_ref, cab1_ref, caw2t_ref, cab2_ref,
                w2t_ref, b2_ref, bn2g_ref, bn2b_ref,
                w3t_ref, b3_ref,
                idx_ref, val_ref):
    rs = 1.0 / jnp.sqrt(1.0 + EPS)
    g = g_ref[...] * (1.0 / S)
    h = jnp.dot(g, w1t_ref[...], preferred_element_type=jnp.float32) + b1_ref[...]
    h = h * (bn1g_ref[...] * rs) + bn1b_ref[...]
    h = _gelu(h)
    t = _gelu(jnp.dot(h, caw1t_ref[...], preferred_element_type=jnp.float32) + cab1_ref[...])
    fc = jnp.dot(t, caw2t_ref[...], preferred_element_type=jnp.float32) + cab2_ref[...]
    att = jax.nn.sigmoid(2.0 * fc)
    h = h * att
    h2 = jnp.dot(h, w2t_ref[...], preferred_element_type=jnp.float32) + b2_ref[...]
    h2 = h2 * (bn2g_ref[...] * rs) + bn2b_ref[...]
    h2 = _gelu(h2)
    scores = jnp.dot(h2, w3t_ref[...], preferred_element_type=jnp.float32) + b3_ref[...]

    ids = jax.lax.broadcasted_iota(jnp.int32, (B, E), 1)
    m1 = jnp.max(scores, axis=1, keepdims=True)
    i1 = jnp.min(jnp.where(scores == m1, ids, E), axis=1, keepdims=True)
    masked = jnp.where(ids == i1, -jnp.inf, scores)
    m2 = jnp.max(masked, axis=1, keepdims=True)
    i2 = jnp.min(jnp.where(masked == m2, ids, E), axis=1, keepdims=True)
    v1 = 1.0 / (1.0 + jnp.exp((m2 - m1) / TEMP))
    v2 = 1.0 - v1
    col = jax.lax.broadcasted_iota(jnp.int32, (B, K), 1)
    idx_ref[...] = jnp.where(col == 0, i1, i2)
    val_ref[...] = jnp.where(col == 0, v1, v2)


def kernel(x, w1, b1, bn1_g, bn1_b, ca_w1, ca_b1, ca_w2, ca_b2, w2, b2, bn2_g, bn2_b, w3, b3):
    x3 = x.reshape(ROWS, H, H)
    gsum = pl.pallas_call(
        _rowsum_kernel,
        in_specs=[pl.BlockSpec(memory_space=pl.ANY)],
        out_specs=pl.BlockSpec((ROWS, 1), lambda: (0, 0)),
        out_shape=jax.ShapeDtypeStruct((ROWS, 1), jnp.float32),
        scratch_shapes=([pltpu.VMEM((RB, H, H), jnp.float32)] * NBUF
                        + [pltpu.SemaphoreType.DMA] * NBUF),
    )(x3)
    g = gsum.reshape(B, C)

    row = lambda v: v.reshape(1, -1)
    full = lambda shp: pl.BlockSpec(shp, lambda: (0,) * len(shp))
    idx, val = pl.pallas_call(
        _mlp_kernel,
        in_specs=[
            full((B, C)),
            full((C, C2)), full((1, C2)), full((1, C2)), full((1, C2)),
            full((C2, R)), full((1, R)), full((R, C2)), full((1, C2)),
            full((C2, C)), full((1, C)), full((1, C)), full((1, C)),
            full((C, E)), full((1, E)),
        ],
        out_specs=[full((B, K)), full((B, K))],
        out_shape=[
            jax.ShapeDtypeStruct((B, K), jnp.int32),
            jax.ShapeDtypeStruct((B, K), jnp.float32),
        ],
    )(g, w1.T, row(b1), row(bn1_g), row(bn1_b),
      ca_w1.T, row(ca_b1), ca_w2.T, row(ca_b2),
      w2.T, row(b2), row(bn2_g), row(bn2_b),
      w3.T, row(b3))
    return idx, val


# speedup vs baseline: 1.0298x; 1.0298x over previous
import jax
import jax.numpy as jnp
from jax.experimental import pallas as pl
from jax.experimental.pallas import tpu as pltpu

B = 8
C = 192
C2 = 2 * C
R = C2 // 16
E = 16
K = 2
TEMP = 2.0
S = 224 * 224
ROWS = B * C
H = 224
RB = 32
NRB = ROWS // RB
EPS = 1e-5


def _gelu(t):
    return 0.5 * t * (1.0 + jax.lax.erf(t * (1.0 / jnp.sqrt(2.0))))


def _rowsum_kernel(x_ref, out_ref):
    s = jnp.sum(x_ref[...], axis=1)
    out_ref[...] = jnp.sum(s, axis=1, keepdims=True)


def _dot_t(a, w):
    return jax.lax.dot_general(a, w, (((1,), (1,)), ((), ())),
                               preferred_element_type=jnp.float32)


def _mlp_kernel(g_ref, w1_ref, b1_ref, bn1g_ref, bn1b_ref,
                caw1_ref, cab1_ref, caw2_ref, cab2_ref,
                w2_ref, b2_ref, bn2g_ref, bn2b_ref,
                w3_ref, b3_ref,
                idx_ref, val_ref):
    rs = 1.0 / jnp.sqrt(1.0 + EPS)
    g = g_ref[...] * (1.0 / S)
    h = _dot_t(g, w1_ref[...]) + b1_ref[...]
    h = h * (bn1g_ref[...] * rs) + bn1b_ref[...]
    h = _gelu(h)
    t = _gelu(_dot_t(h, caw1_ref[...]) + cab1_ref[...])
    fc = _dot_t(t, caw2_ref[...]) + cab2_ref[...]
    att = jax.nn.sigmoid(2.0 * fc)
    h = h * att
    h2 = _dot_t(h, w2_ref[...]) + b2_ref[...]
    h2 = h2 * (bn2g_ref[...] * rs) + bn2b_ref[...]
    h2 = _gelu(h2)
    scores = _dot_t(h2, w3_ref[...]) + b3_ref[...]

    ids = jax.lax.broadcasted_iota(jnp.int32, (B, E), 1)
    m1 = jnp.max(scores, axis=1, keepdims=True)
    i1 = jnp.min(jnp.where(scores == m1, ids, E), axis=1, keepdims=True)
    masked = jnp.where(ids == i1, -jnp.inf, scores)
    m2 = jnp.max(masked, axis=1, keepdims=True)
    i2 = jnp.min(jnp.where(masked == m2, ids, E), axis=1, keepdims=True)
    v1 = 1.0 / (1.0 + jnp.exp((m2 - m1) / TEMP))
    v2 = 1.0 - v1
    col = jax.lax.broadcasted_iota(jnp.int32, (B, K), 1)
    idx_ref[...] = jnp.where(col == 0, i1, i2)
    val_ref[...] = jnp.where(col == 0, v1, v2)


def kernel(x, w1, b1, bn1_g, bn1_b, ca_w1, ca_b1, ca_w2, ca_b2, w2, b2, bn2_g, bn2_b, w3, b3):
    x3 = x.reshape(ROWS, H, H)
    gsum = pl.pallas_call(
        _rowsum_kernel,
        grid=(NRB,),
        in_specs=[pl.BlockSpec((RB, H, H), lambda i: (i, 0, 0))],
        out_specs=pl.BlockSpec((RB, 1), lambda i: (i, 0)),
        out_shape=jax.ShapeDtypeStruct((ROWS, 1), jnp.float32),
        compiler_params=pltpu.CompilerParams(
            dimension_semantics=("arbitrary",),
        ),
    )(x3)
    g = gsum.reshape(B, C)

    row = lambda v: v.reshape(1, -1)
    full = lambda shp: pl.BlockSpec(shp, lambda: (0,) * len(shp))
    idx, val = pl.pallas_call(
        _mlp_kernel,
        in_specs=[
            full((B, C)),
            full((C2, C)), full((1, C2)), full((1, C2)), full((1, C2)),
            full((R, C2)), full((1, R)), full((C2, R)), full((1, C2)),
            full((C, C2)), full((1, C)), full((1, C)), full((1, C)),
            full((E, C)), full((1, E)),
        ],
        out_specs=[full((B, K)), full((B, K))],
        out_shape=[
            jax.ShapeDtypeStruct((B, K), jnp.int32),
            jax.ShapeDtypeStruct((B, K), jnp.float32),
        ],
    )(g, w1, row(b1), row(bn1_g), row(bn1_b),
      ca_w1, row(ca_b1), ca_w2, row(ca_b2),
      w2, row(b2), row(bn2_g), row(bn2_b),
      w3, row(b3))
    return idx, val
